# baseline (speedup 1.0000x reference)
import functools

import jax
import jax.numpy as jnp
from jax import lax
from jax.experimental import pallas as pl
from jax.experimental.pallas import tpu as pltpu
from jax.experimental.pallas import tpu_sc as plsc

B = 4096
N_NUM = 13
N_CAT = 26
CARD = 1000
D = 64
N_FEAT = N_NUM + N_CAT
FPAD = 128
KDIM = N_CAT * D
TB = 1024

SPW = 128
CH = 16
NCHUNK = SPW // CH
ROWS_CH = CH * N_CAT
QROWS = 104
NQ = ROWS_CH // QROWS


def _sc_gather_reduce(xcat_flat, m2, emb_table):
    info = plsc.get_sparse_core_info()
    nc, ns = info.num_cores, info.num_subcores
    nw = nc * ns
    rpw = SPW * N_CAT

    mesh = plsc.VectorSubcoreMesh(core_axis_name="c", subcore_axis_name="s")

    @functools.partial(
        pl.kernel,
        mesh=mesh,
        out_type=jax.ShapeDtypeStruct((B, 2 * D), jnp.float32),
        compiler_params=pltpu.CompilerParams(use_tc_tiling_on_sc=False),
        scratch_types=[
            pltpu.VMEM((rpw,), jnp.int32),
            pltpu.VMEM((3, ROWS_CH, D), jnp.float32),
            pltpu.VMEM((SPW, D), jnp.float32),
            pltpu.VMEM((SPW, 2 * D), jnp.float32),
            pltpu.SemaphoreType.DMA,
            pltpu.SemaphoreType.DMA,
            pltpu.SemaphoreType.DMA,
        ],
    )
    def k(xcat_hbm, m2_hbm, table_hbm, out_hbm,
          idx_v, rows_v, m_v, out_v, sem0, sem1, sem2):
        wid = lax.axis_index("s") * nc + lax.axis_index("c")
        base = wid * rpw
        sbase = wid * SPW
        pltpu.sync_copy(xcat_hbm.at[pl.ds(base, rpw)], idx_v)
        pltpu.sync_copy(m2_hbm.at[pl.ds(sbase, SPW)], m_v)

        lane = lax.iota(jnp.int32, 16)

        def add_off(i, _):
            v = idx_v[pl.ds(i * 16, 16)]
            p = base + i * 16 + lane
            f = lax.rem(p, N_CAT)
            idx_v[pl.ds(i * 16, 16)] = v + f * CARD
            return 0

        lax.fori_loop(0, rpw // 16, add_off, 0)

        sems = (sem0, sem1, sem2)

        def start(c):
            buf = c % 3
            return [
                pltpu.async_copy(
                    table_hbm.at[idx_v.at[pl.ds(c * ROWS_CH + q * QROWS,
                                                QROWS)]],
                    rows_v.at[buf, pl.ds(q * QROWS, QROWS)],
                    sems[buf])
                for q in range(NQ)
            ]

        def compute_chunk(c):
            buf = c % 3

            def sbody(sl, _):
                s = c * CH + sl
                mrow = [m_v[s, pl.ds(t * 16, 16)] for t in range(4)]
                rbase = sl * N_CAT
                z = jnp.zeros((16,), jnp.float32)
                accn = [z, z, z, z]
                accc = [z, z, z, z]
                for j in range(N_CAT):
                    wn = mrow[j // 16][j % 16]
                    wc = mrow[(N_CAT + j) // 16][(N_CAT + j) % 16]
                    for kk in range(4):
                        row = rows_v[buf, rbase + j, pl.ds(kk * 16, 16)]
                        accn[kk] = accn[kk] + wn * row
                        accc[kk] = accc[kk] + wc * row
                for kk in range(4):
                    out_v[s, pl.ds(kk * 16, 16)] = accn[kk]
                    out_v[s, pl.ds(D + kk * 16, 16)] = accc[kk]
                return 0

            lax.fori_loop(0, CH, sbody, 0)

        pend = {0: start(0), 1: start(1)}
        for c in range(NCHUNK):
            if c + 2 < NCHUNK:
                pend[c + 2] = start(c + 2)
            for h in pend.pop(c):
                h.wait()
            compute_chunk(c)
        pltpu.sync_copy(out_v, out_hbm.at[pl.ds(sbase, SPW)])

    return k(xcat_flat, m2, emb_table)


def _tc_body(xn_ref, mn_ref, mc_ref, g_ref, idxn_ref, idxc_ref,
             wnum_ref, bnum_ref, bcat_ref, dnw_ref, dnb_ref,
             wflat_ref, dcb_ref,
             pn_ref, pc_ref, m_ref):
    xn = xn_ref[...]
    mn = mn_ref[...]
    mc = mc_ref[...]
    g = g_ref[...]

    wnum = wnum_ref[...]
    bias = jnp.concatenate([bnum_ref[...], bcat_ref[...]], axis=0)

    sn = jnp.dot(mn[:, :N_NUM] * xn, wnum, preferred_element_type=jnp.float32)
    sn = sn + jnp.dot(mn, bias, preferred_element_type=jnp.float32)
    sn = sn + g[:, :D]
    sc_ = jnp.dot(mc[:, :N_NUM] * xn, wnum, preferred_element_type=jnp.float32)
    sc_ = sc_ + jnp.dot(mc, bias, preferred_element_type=jnp.float32)
    sc_ = sc_ + g[:, D:]

    den_n = jnp.sum(mn, axis=1, keepdims=True)
    den_c = jnp.sum(mc, axis=1, keepdims=True)
    en = sn / den_n
    ec = sc_ / den_c

    idxn = idxn_ref[...]
    pn_all = lax.dot_general(en, dnw_ref[...], (((1,), (1,)), ((), ())),
                             preferred_element_type=jnp.float32)
    col = lax.broadcasted_iota(jnp.int32, (TB, N_NUM), 1)
    ohn = jnp.where(col == idxn, 1.0, 0.0)
    pn = jnp.sum(ohn * pn_all, axis=1, keepdims=True)
    pn_ref[...] = pn + jnp.dot(ohn, dnb_ref[...],
                               preferred_element_type=jnp.float32)

    idxc = jnp.mod(idxc_ref[...] - N_NUM, N_CAT)
    ecb = ec.astype(jnp.bfloat16)
    zb = jnp.zeros_like(ecb)
    for jj in range(N_CAT // 2):
        left = jnp.where(idxc == 2 * jj, ecb, zb)
        right = jnp.where(idxc == 2 * jj + 1, ecb, zb)
        m_ref[:, jj * 2 * D:(jj + 1) * 2 * D] = jnp.concatenate(
            [left, right], axis=1)
    t26 = lax.broadcasted_iota(jnp.int32, (TB, N_CAT), 1)
    oh = jnp.where(t26 == idxc, 1.0, 0.0)
    pc = jnp.dot(m_ref[...], wflat_ref[...], preferred_element_type=jnp.float32)
    pc_ref[...] = pc + jnp.dot(oh, dcb_ref[...],
                               preferred_element_type=jnp.float32)


def _tc_forward(xn, mn, mc, g, idxn, idxc, wnum, bnum, bcat, dnw, dnb,
                wflat, dcb):
    grid = (B // TB,)
    row = lambda i: (i, 0)
    const = lambda i: (0, 0)
    return pl.pallas_call(
        _tc_body,
        grid=grid,
        in_specs=[
            pl.BlockSpec((TB, N_NUM), row),
            pl.BlockSpec((TB, N_FEAT), row),
            pl.BlockSpec((TB, N_FEAT), row),
            pl.BlockSpec((TB, 2 * D), row),
            pl.BlockSpec((TB, 1), row),
            pl.BlockSpec((TB, 1), row),
            pl.BlockSpec((N_NUM, D), const),
            pl.BlockSpec((N_NUM, D), const),
            pl.BlockSpec((N_CAT, D), const),
            pl.BlockSpec((N_NUM, D), const),
            pl.BlockSpec((N_NUM, 1), const),
            pl.BlockSpec((KDIM, CARD), const),
            pl.BlockSpec((N_CAT, CARD), const),
        ],
        out_specs=[
            pl.BlockSpec((TB, 1), row),
            pl.BlockSpec((TB, CARD), row),
        ],
        out_shape=[
            jax.ShapeDtypeStruct((B, 1), jnp.float32),
            jax.ShapeDtypeStruct((B, CARD), jnp.float32),
        ],
        scratch_shapes=[pltpu.VMEM((TB, KDIM), jnp.bfloat16)],
        compiler_params=pltpu.CompilerParams(
            dimension_semantics=("parallel",)),
    )(xn, mn, mc, g, idxn, idxc, wnum, bnum, bcat, dnw, dnb, wflat, dcb)


def kernel(x_num, x_cat, mask_num, mask_cat, pred_idx_num, pred_idx_cat,
           W_num, b_num, emb_table, b_cat,
           dec_num_w, dec_num_b, dec_cat_w, dec_cat_b):
    f32 = jnp.float32
    xcat_flat = x_cat.astype(jnp.int32).reshape(B * N_CAT)
    m2 = (jnp.zeros((B, D), f32)
          .at[:, :N_CAT].set(mask_num[:, N_NUM:])
          .at[:, N_CAT:2 * N_CAT].set(mask_cat[:, N_NUM:]))
    g = _sc_gather_reduce(xcat_flat, m2, emb_table.astype(f32))

    pn, pc = _tc_forward(
        x_num, mask_num, mask_cat, g,
        pred_idx_num.astype(jnp.int32), pred_idx_cat.astype(jnp.int32),
        W_num, b_num, b_cat,
        dec_num_w.reshape(N_NUM, D), dec_num_b,
        dec_cat_w.reshape(KDIM, CARD).astype(jnp.bfloat16), dec_cat_b)
    return (pn, pc)


# speedup vs baseline: 1.0143x; 1.0143x over previous
import functools

import jax
import jax.numpy as jnp
from jax import lax
from jax.experimental import pallas as pl
from jax.experimental.pallas import tpu as pltpu
from jax.experimental.pallas import tpu_sc as plsc

B = 4096
N_NUM = 13
N_CAT = 26
CARD = 1000
D = 64
N_FEAT = N_NUM + N_CAT
KDIM = N_CAT * D
TB = 1024

SPW = 128
CH = 16
NCHUNK = SPW // CH
ROWS_CH = CH * N_CAT
QROWS = 104
NQ = ROWS_CH // QROWS


def _sc_gather_reduce(xcat_flat, m2, emb_table):
    info = plsc.get_sparse_core_info()
    nc, ns = info.num_cores, info.num_subcores
    nw = nc * ns
    rpw = SPW * N_CAT

    mesh = plsc.VectorSubcoreMesh(core_axis_name="c", subcore_axis_name="s")

    @functools.partial(
        pl.kernel,
        mesh=mesh,
        out_type=jax.ShapeDtypeStruct((B, 2 * D), jnp.float32),
        compiler_params=pltpu.CompilerParams(use_tc_tiling_on_sc=False),
        scratch_types=[
            pltpu.VMEM((rpw,), jnp.int32),
            pltpu.VMEM((3, ROWS_CH, D), jnp.float32),
            pltpu.VMEM((SPW, 2 * D), jnp.float32),
            pltpu.VMEM((SPW, 2 * D), jnp.float32),
            pltpu.SemaphoreType.DMA,
            pltpu.SemaphoreType.DMA,
            pltpu.SemaphoreType.DMA,
        ],
    )
    def k(xcat_hbm, m2_hbm, table_hbm, out_hbm,
          idx_v, rows_v, m_v, out_v, sem0, sem1, sem2):
        wid = lax.axis_index("s") * nc + lax.axis_index("c")
        base = wid * rpw
        sbase = wid * SPW
        pltpu.sync_copy(xcat_hbm.at[pl.ds(base, rpw)], idx_v)
        pltpu.sync_copy(m2_hbm.at[pl.ds(sbase, SPW)], m_v)

        lane = lax.iota(jnp.int32, 16)

        def add_off(i, _):
            v = idx_v[pl.ds(i * 16, 16)]
            p = base + i * 16 + lane
            f = lax.rem(p, N_CAT)
            idx_v[pl.ds(i * 16, 16)] = v + f * CARD
            return 0

        lax.fori_loop(0, rpw // 16, add_off, 0)

        sems = (sem0, sem1, sem2)

        def start(c):
            buf = c % 3
            return [
                pltpu.async_copy(
                    table_hbm.at[idx_v.at[pl.ds(c * ROWS_CH + q * QROWS,
                                                QROWS)]],
                    rows_v.at[buf, pl.ds(q * QROWS, QROWS)],
                    sems[buf])
                for q in range(NQ)
            ]

        def compute_chunk(c):
            buf = c % 3

            def sbody(sl, _):
                s = c * CH + sl
                mrow = [m_v[s, pl.ds(t * 16, 16)] for t in range(4)]
                rbase = sl * N_CAT
                z = jnp.zeros((16,), jnp.float32)
                accn = [z, z, z, z]
                accc = [z, z, z, z]
                for j in range(N_CAT):
                    wn = mrow[j // 16][j % 16]
                    wc = mrow[(N_CAT + j) // 16][(N_CAT + j) % 16]
                    for kk in range(4):
                        row = rows_v[buf, rbase + j, pl.ds(kk * 16, 16)]
                        accn[kk] = accn[kk] + wn * row
                        accc[kk] = accc[kk] + wc * row
                for kk in range(4):
                    out_v[s, pl.ds(kk * 16, 16)] = accn[kk]
                    out_v[s, pl.ds(D + kk * 16, 16)] = accc[kk]
                return 0

            lax.fori_loop(0, CH, sbody, 0)

        pend = {0: start(0), 1: start(1)}
        for c in range(NCHUNK):
            if c + 2 < NCHUNK:
                pend[c + 2] = start(c + 2)
            for h in pend.pop(c):
                h.wait()
            compute_chunk(c)
        pltpu.sync_copy(out_v, out_hbm.at[pl.ds(sbase, SPW)])

    return k(xcat_flat, m2, emb_table)


def _tc_body(xn_ref, mn_ref, mc_ref, g_ref, idxn_ref, idxc_ref,
             wnum_ref, bnum_ref, bcat_ref, dnw_ref, dnb_ref,
             wflat_ref, dcb_ref,
             pn_ref, pc_ref, m_ref):
    xn = xn_ref[...]
    mn = mn_ref[...]
    mc = mc_ref[...]
    g = g_ref[...]

    wnum = wnum_ref[...]
    bias = jnp.concatenate([bnum_ref[...], bcat_ref[...]], axis=0)

    sn = jnp.dot(mn[:, :N_NUM] * xn, wnum, preferred_element_type=jnp.float32)
    sn = sn + jnp.dot(mn, bias, preferred_element_type=jnp.float32)
    sn = sn + g[:, :D]
    sc_ = jnp.dot(mc[:, :N_NUM] * xn, wnum, preferred_element_type=jnp.float32)
    sc_ = sc_ + jnp.dot(mc, bias, preferred_element_type=jnp.float32)
    sc_ = sc_ + g[:, D:]

    den_n = jnp.sum(mn, axis=1, keepdims=True)
    den_c = jnp.sum(mc, axis=1, keepdims=True)
    en = sn / den_n
    ec = sc_ / den_c

    idxn = idxn_ref[...]
    pn_all = lax.dot_general(en, dnw_ref[...], (((1,), (1,)), ((), ())),
                             preferred_element_type=jnp.float32)
    col = lax.broadcasted_iota(jnp.int32, (TB, N_NUM), 1)
    ohn = jnp.where(col == idxn, 1.0, 0.0)
    pn = jnp.sum(ohn * pn_all, axis=1, keepdims=True)
    pn_ref[...] = pn + jnp.dot(ohn, dnb_ref[...],
                               preferred_element_type=jnp.float32)

    idxc = jnp.mod(idxc_ref[...] - N_NUM, N_CAT)
    ecb = ec.astype(jnp.bfloat16)
    zb = jnp.zeros_like(ecb)
    for jj in range(N_CAT // 2):
        left = jnp.where(idxc == 2 * jj, ecb, zb)
        right = jnp.where(idxc == 2 * jj + 1, ecb, zb)
        m_ref[:, jj * 2 * D:(jj + 1) * 2 * D] = jnp.concatenate(
            [left, right], axis=1)
    t26 = lax.broadcasted_iota(jnp.int32, (TB, N_CAT), 1)
    oh = jnp.where(t26 == idxc, 1.0, 0.0)
    pc = jnp.dot(m_ref[...], wflat_ref[...], preferred_element_type=jnp.float32)
    pc_ref[...] = pc + jnp.dot(oh, dcb_ref[...],
                               preferred_element_type=jnp.float32)


def _tc_forward(xn, mn, mc, g, idxn, idxc, wnum, bnum, bcat, dnw, dnb,
                wflat, dcb):
    grid = (B // TB,)
    row = lambda i: (i, 0)
    const = lambda i: (0, 0)
    return pl.pallas_call(
        _tc_body,
        grid=grid,
        in_specs=[
            pl.BlockSpec((TB, N_NUM), row),
            pl.BlockSpec((TB, N_FEAT), row),
            pl.BlockSpec((TB, N_FEAT), row),
            pl.BlockSpec((TB, 2 * D), row),
            pl.BlockSpec((TB, 1), row),
            pl.BlockSpec((TB, 1), row),
            pl.BlockSpec((N_NUM, D), const),
            pl.BlockSpec((N_NUM, D), const),
            pl.BlockSpec((N_CAT, D), const),
            pl.BlockSpec((N_NUM, D), const),
            pl.BlockSpec((N_NUM, 1), const),
            pl.BlockSpec((KDIM, CARD), const),
            pl.BlockSpec((N_CAT, CARD), const),
        ],
        out_specs=[
            pl.BlockSpec((TB, 1), row),
            pl.BlockSpec((TB, CARD), row),
        ],
        out_shape=[
            jax.ShapeDtypeStruct((B, 1), jnp.float32),
            jax.ShapeDtypeStruct((B, CARD), jnp.float32),
        ],
        scratch_shapes=[pltpu.VMEM((TB, KDIM), jnp.bfloat16)],
        compiler_params=pltpu.CompilerParams(
            dimension_semantics=("parallel",)),
    )(xn, mn, mc, g, idxn, idxc, wnum, bnum, bcat, dnw, dnb, wflat, dcb)


def kernel(x_num, x_cat, mask_num, mask_cat, pred_idx_num, pred_idx_cat,
           W_num, b_num, emb_table, b_cat,
           dec_num_w, dec_num_b, dec_cat_w, dec_cat_b):
    f32 = jnp.float32
    xcat_flat = x_cat.astype(jnp.int32).reshape(B * N_CAT)
    m2 = (jnp.zeros((B, 2 * D), f32)
          .at[:, :N_CAT].set(mask_num[:, N_NUM:])
          .at[:, N_CAT:2 * N_CAT].set(mask_cat[:, N_NUM:]))
    g = _sc_gather_reduce(xcat_flat, m2, emb_table.astype(f32))

    pn, pc = _tc_forward(
        x_num, mask_num, mask_cat, g,
        pred_idx_num.astype(jnp.int32), pred_idx_cat.astype(jnp.int32),
        W_num, b_num, b_cat,
        dec_num_w.reshape(N_NUM, D), dec_num_b,
        dec_cat_w.reshape(KDIM, CARD).astype(jnp.bfloat16), dec_cat_b)
    return (pn, pc)
